# baseline (speedup 1.0000x reference)
import functools

import jax
import jax.numpy as jnp
from jax import lax
from jax.experimental import pallas as pl
from jax.experimental.pallas import tpu as pltpu
from jax.experimental.pallas import tpu_sc as plsc

_N = 10000
_D = 128
_E = 320000
_NC = 2
_NS = 16
_CH = 128
_NBUF = 2
_NIDX = 4
_NPW = _NIDX * (-(-_E // (_CH * _NS * _NIDX)))
_EPAD = _NPW * _CH * _NS
_RT = 632
_NA = _RT * _NS
_HPAD = 8


def _sc_two_sided_accumulate(h_pad, eidx, zero_blk):
    mesh = plsc.VectorSubcoreMesh(core_axis_name="c", subcore_axis_name="s")

    @functools.partial(
        pl.kernel,
        out_type=jax.ShapeDtypeStruct((_NC, _NA, _D), jnp.float32),
        mesh=mesh,
        scratch_types=[
            *[pltpu.VMEM((_CH,), jnp.int32) for _ in range(_NIDX)],
            *[pltpu.VMEM((_CH,), jnp.int32) for _ in range(_NIDX)],
            *[pltpu.VMEM((_CH, _D), jnp.float32) for _ in range(_NBUF)],
            pltpu.VMEM_SHARED((_NA, _D), jnp.float32),
            *[pltpu.SemaphoreType.DMA for _ in range(2 * _NIDX + 2 * _NBUF)],
        ],
    )
    def k(h_hbm, e_hbm, z_hbm, out_hbm, *rest):
        ig = rest[:_NIDX]
        idd = rest[_NIDX:2 * _NIDX]
        rows = rest[2 * _NIDX:2 * _NIDX + _NBUF]
        acc = rest[2 * _NIDX + _NBUF]
        sems = rest[2 * _NIDX + _NBUF + 1:]
        isg = sems[:_NIDX]
        isd = sems[_NIDX:2 * _NIDX]
        gsem = sems[2 * _NIDX:2 * _NIDX + _NBUF]
        ssem = sems[2 * _NIDX + _NBUF:]
        c = lax.axis_index("c")
        s = lax.axis_index("s")

        def idx_start(j, q):
            pltpu.async_copy(e_hbm.at[c, s, j], ig[q], isg[q])
            pltpu.async_copy(e_hbm.at[1, s, j], idd[q], isd[q])

        def idx_wait(j, q):
            pltpu.make_async_copy(e_hbm.at[c, s, j], ig[q], isg[q]).wait()
            pltpu.make_async_copy(e_hbm.at[1, s, j], idd[q], isd[q]).wait()

        def gather_start(j, q, b):
            pltpu.async_copy(h_hbm.at[ig[q]], rows[b], gsem[b])

        def gather_wait(j, q, b):
            pltpu.make_async_copy(h_hbm.at[ig[q]], rows[b], gsem[b]).wait()

        def scatter_start(j, q, b):
            pltpu.async_copy(rows[b], acc.at[idd[q]], ssem[b], add=True)

        def scatter_wait(j, q, b):
            pltpu.make_async_copy(rows[b], acc.at[idd[q]], ssem[b]).wait()

        for q in range(_NIDX):
            idx_start(q, q)
        pltpu.sync_copy(z_hbm, acc.at[pl.ds(s * _RT, _RT)])
        idx_wait(0, 0)
        gather_start(0, 0, 0)
        plsc.subcore_barrier()

        def body(i, carry):
            for u in range(_NIDX):
                j = i * _NIDX + u
                b = u % _NBUF
                gather_wait(j, u, b)
                scatter_start(j, u, b)

                @pl.when(j >= 1)
                def _():
                    scatter_wait(j - 1, (u - 1) % _NIDX, 1 - b)

                @pl.when(j + 3 < _NPW)
                def _():
                    idx_start(j + 3, (u + 3) % _NIDX)

                @pl.when(j + 1 < _NPW)
                def _():
                    idx_wait(j + 1, (u + 1) % _NIDX)
                    gather_start(j + 1, (u + 1) % _NIDX, 1 - b)
            return carry

        lax.fori_loop(0, _NPW // _NIDX, body, 0)
        scatter_wait(_NPW - 1, (_NPW - 1) % _NIDX, (_NPW - 1) % _NBUF)
        plsc.subcore_barrier()

        pltpu.sync_copy(acc.at[pl.ds(s * _RT, _RT)],
                        out_hbm.at[c, pl.ds(s * _RT, _RT)])

    return k(h_pad, eidx, zero_blk)


_BN = 400


def _tc_finalize(partials):
    def body(p_ref, o_ref):
        agg = p_ref[0] - p_ref[1]
        ss = jnp.sum(agg * agg, axis=1, keepdims=True)
        o_ref[...] = agg / (jnp.sqrt(ss) + 1e-7)

    return pl.pallas_call(
        body,
        grid=(_N // _BN,),
        in_specs=[pl.BlockSpec((_NC, _BN, _D), lambda i: (0, i, 0))],
        out_specs=pl.BlockSpec((_BN, _D), lambda i: (i, 0)),
        out_shape=jax.ShapeDtypeStruct((_N, _D), jnp.float32),
    )(partials)


def kernel(h, edge_index, r):
    eidx = jnp.concatenate(
        [edge_index.astype(jnp.int32),
         jnp.full((2, _EPAD - _E), _N, jnp.int32)], axis=1)
    eidx = eidx.reshape(2, _NS, _NPW, _CH)
    h_pad = jnp.concatenate(
        [h, jnp.zeros((_HPAD, _D), jnp.float32)], axis=0)
    zero_blk = jnp.zeros((_RT, _D), jnp.float32)
    partials = _sc_two_sided_accumulate(h_pad, eidx, zero_blk)
    return _tc_finalize(partials)
